# baseline (speedup 1.0000x reference)
import functools

import jax
import jax.numpy as jnp
from jax import lax
from jax.experimental import pallas as pl
from jax.experimental.pallas import tpu as pltpu
from jax.experimental.pallas import tpu_sc as plsc

N = 10000
E = 320000
HEADS = 8
OUT_F = 16
HF = HEADS * OUT_F

NC = 2
NS = 16
NW = NC * NS

f32 = jnp.float32
i32 = jnp.int32

_mesh = plsc.VectorSubcoreMesh(core_axis_name="c", subcore_axis_name="s")

_BN = 1000


def _p1_body(x_ref, wt_ref, a_ref, h_ref, as_ref):
    h = jnp.dot(x_ref[...], wt_ref[...], preferred_element_type=f32)
    h_ref[...] = h
    as_ref[...] = jnp.dot(h, a_ref[...], preferred_element_type=f32)


def _p1(x, wt, aexp):
    return pl.pallas_call(
        _p1_body,
        grid=(N // _BN,),
        in_specs=[
            pl.BlockSpec((_BN, HF), lambda i: (i, 0)),
            pl.BlockSpec((HF, HF), lambda i: (0, 0)),
            pl.BlockSpec((HF, 16), lambda i: (0, 0)),
        ],
        out_specs=[
            pl.BlockSpec((_BN, HF), lambda i: (i, 0)),
            pl.BlockSpec((_BN, 16), lambda i: (i, 0)),
        ],
        out_shape=[
            jax.ShapeDtypeStruct((N, HF), f32),
            jax.ShapeDtypeStruct((N, 16), f32),
        ],
    )(x, wt, aexp)


_CA = 512
_NCH_A = E // _CA
_ITER_A = (_NCH_A + NW - 1) // NW


@functools.partial(
    pl.kernel,
    out_type=(
        jax.ShapeDtypeStruct((E, HF), f32),
        jax.ShapeDtypeStruct((E, 16), f32),
    ),
    mesh=_mesh,
    compiler_params=pltpu.CompilerParams(use_tc_tiling_on_sc=False, needs_layout_passes=False),
    scratch_types=[
        pltpu.VMEM((_CA,), i32),
        pltpu.VMEM((_CA,), i32),
        pltpu.VMEM((_CA, HF), f32),
        pltpu.VMEM((_CA, 16), f32),
        pltpu.SemaphoreType.DMA,
        pltpu.SemaphoreType.DMA,
    ],
)
def _p2(h_hbm, as_hbm, src_hbm, dst_hbm, hd_out, ase_out,
        di_v, si_v, hd_v, ase_v, sem1, sem2):
    wid = lax.axis_index("s") * NC + lax.axis_index("c")

    def chunk(ci, carry):
        ck = wid + NW * ci

        @pl.when(ck < _NCH_A)
        def _():
            base = ck * _CA
            pltpu.sync_copy(dst_hbm.at[pl.ds(base, _CA)], di_v)
            pltpu.sync_copy(src_hbm.at[pl.ds(base, _CA)], si_v)
            for g in range(_CA // 128):
                sl = pl.ds(g * 128, 128)
                pltpu.async_copy(h_hbm.at[di_v.at[sl]], hd_v.at[sl], sem1)
                pltpu.async_copy(as_hbm.at[si_v.at[sl]], ase_v.at[sl], sem2)
            for g in range(_CA // 128):
                sl = pl.ds(g * 128, 128)
                pltpu.make_async_copy(h_hbm.at[di_v.at[sl]], hd_v.at[sl], sem1).wait()
                pltpu.make_async_copy(as_hbm.at[si_v.at[sl]], ase_v.at[sl], sem2).wait()
            pltpu.sync_copy(hd_v, hd_out.at[pl.ds(base, _CA)])
            pltpu.sync_copy(ase_v, ase_out.at[pl.ds(base, _CA)])

        return carry

    lax.fori_loop(0, _ITER_A, chunk, None)


_BE = 2000


def _p3_body(ea_ref, hd_ref, ase_ref, wet_ref, be_ref, adst_ref, s_ref, out_ref):
    enc = jnp.maximum(
        jnp.dot(ea_ref[...], wet_ref[...], preferred_element_type=f32)
        + be_ref[...], 0.0)
    hd = hd_ref[...]
    aenc = jnp.dot(enc * hd, s_ref[...], preferred_element_type=f32)
    ad = jnp.dot(hd, adst_ref[...], preferred_element_type=f32)
    alpha = ase_ref[...] + aenc + ad
    alpha = jnp.where(alpha > 0, alpha, 0.2 * alpha)
    out_ref[...] = jnp.exp(alpha)


def _p3(edge_attr, hd, ase, wet, be2, adst, smat):
    return pl.pallas_call(
        _p3_body,
        grid=(E // _BE,),
        in_specs=[
            pl.BlockSpec((_BE, 16), lambda i: (i, 0)),
            pl.BlockSpec((_BE, HF), lambda i: (i, 0)),
            pl.BlockSpec((_BE, 16), lambda i: (i, 0)),
            pl.BlockSpec((16, HF), lambda i: (0, 0)),
            pl.BlockSpec((1, HF), lambda i: (0, 0)),
            pl.BlockSpec((HF, 16), lambda i: (0, 0)),
            pl.BlockSpec((HF, 16), lambda i: (0, 0)),
        ],
        out_specs=pl.BlockSpec((_BE, 16), lambda i: (i, 0)),
        out_shape=jax.ShapeDtypeStruct((E, 16), f32),
    )(edge_attr, hd, ase, wet, be2, adst, smat)


_C6 = 64
_K6 = 4
_KI = 8
_ESC = E // NC
_NCH_6 = _ESC // _C6
_CMAX = -(-_NCH_6 // NS)
_NOCT = (_CMAX + _KI - 1) // _KI
_RT = N // NS
_NG = _C6 // 16


@functools.partial(
    pl.kernel,
    out_type=(
        jax.ShapeDtypeStruct((NC, N, 16), f32),
        jax.ShapeDtypeStruct((NC, N, HF), f32),
    ),
    mesh=_mesh,
    compiler_params=pltpu.CompilerParams(use_tc_tiling_on_sc=False, needs_layout_passes=False),
    scratch_types=[
        pltpu.VMEM((_KI, _C6), i32),
        pltpu.VMEM((_KI, _C6), i32),
        pltpu.VMEM((_K6, _C6, HF), f32),
        pltpu.VMEM((_K6, _C6, 16), f32),
        pltpu.VMEM_SHARED((N, 16), f32),
        pltpu.VMEM_SHARED((N, HF), f32),
        pltpu.SemaphoreType.DMA,
        pltpu.SemaphoreType.DMA,
        pltpu.SemaphoreType.DMA,
        pltpu.SemaphoreType.DMA,
        pltpu.SemaphoreType.DMA,
    ],
)
def _p6m(h_hbm, exp_hbm, src_hbm, dst_hbm, z16_hbm, z128_hbm,
         p_out, agg_out,
         si_v, di_v, hs_v, e_v,
         acc16, acc128, sem_i, sem_h, sem_e, sem_s16, sem_s128):
    cid = lax.axis_index("c")
    sid = lax.axis_index("s")
    rbase = sid * _RT
    pltpu.sync_copy(z16_hbm.at[pl.ds(rbase, _RT)], acc16.at[pl.ds(rbase, _RT)])
    pltpu.sync_copy(z128_hbm.at[pl.ds(rbase, _RT)], acc128.at[pl.ds(rbase, _RT)])
    plsc.subcore_barrier()

    def ck_of(c):
        return sid + NS * c

    def valid(c):
        return ck_of(c) < _NCH_6

    def fire_idx(c, s8):
        base = cid * _ESC + ck_of(c) * _C6
        pltpu.async_copy(src_hbm.at[pl.ds(base, _C6)], si_v.at[s8], sem_i)
        pltpu.async_copy(dst_hbm.at[pl.ds(base, _C6)], di_v.at[s8], sem_i)

    def wait_idx(s8):
        pltpu.make_async_copy(src_hbm.at[pl.ds(0, _C6)], si_v.at[s8], sem_i).wait()
        pltpu.make_async_copy(dst_hbm.at[pl.ds(0, _C6)], di_v.at[s8], sem_i).wait()

    def fire_main(c, s8, s4):
        base = cid * _ESC + ck_of(c) * _C6
        pltpu.async_copy(h_hbm.at[si_v.at[s8]], hs_v.at[s4], sem_h)
        pltpu.async_copy(exp_hbm.at[pl.ds(base, _C6)], e_v.at[s4], sem_e)

    def wait_main(s8, s4):
        pltpu.make_async_copy(h_hbm.at[si_v.at[s8]], hs_v.at[s4], sem_h).wait()
        pltpu.make_async_copy(exp_hbm.at[pl.ds(0, _C6)], e_v.at[s4], sem_e).wait()

    def wait_scatters(s4):
        pltpu.make_async_copy(e_v.at[s4], acc16.at[di_v.at[0]], sem_s16).wait()
        pltpu.make_async_copy(hs_v.at[s4], acc128.at[di_v.at[0]], sem_s128).wait()

    def process(s8, s4):
        pltpu.async_copy(e_v.at[s4], acc16.at[di_v.at[s8]], sem_s16, add=True)

        def edge(ei, carry2):
            ev_row = e_v.at[s4][ei, :]
            for hh_ in range(HEADS):
                w = ev_row[jnp.full((16,), hh_, i32)]
                sl = pl.ds(hh_ * OUT_F, OUT_F)
                hs_v.at[s4][ei, sl] = hs_v.at[s4][ei, sl] * w
            return carry2

        lax.fori_loop(0, _C6, edge, None)
        pltpu.async_copy(hs_v.at[s4], acc128.at[di_v.at[s8]], sem_s128, add=True)

    fire_idx(0, 0)
    fire_idx(1, 1)
    wait_idx(0)
    fire_main(0, 0, 0)

    def octet(j, carry):
        for par in range(_KI):
            c = _KI * j + par
            s8 = par
            n8 = (par + 1) % _KI
            s4 = par % _K6
            n4 = (par + 1) % _K6

            @pl.when(valid(c + 1))
            def _():
                wait_idx(n8)

                @pl.when(c + 1 >= _K6)
                def _():
                    wait_scatters(n4)

                fire_main(c + 1, n8, n4)

            @pl.when(valid(c))
            def _():
                wait_main(s8, s4)
                process(s8, s4)

            @pl.when(valid(c + 2))
            def _():
                fire_idx(c + 2, (par + 2) % _KI)

        return carry

    lax.fori_loop(0, _NOCT, octet, None)
    for s4 in range(_K6):
        @pl.when(ck_of(s4) < _NCH_6)
        def _():
            wait_scatters(s4)

    plsc.subcore_barrier()
    pltpu.sync_copy(acc16.at[pl.ds(rbase, _RT)],
                    p_out.at[cid].at[pl.ds(rbase, _RT)])
    pltpu.sync_copy(acc128.at[pl.ds(rbase, _RT)],
                    agg_out.at[cid].at[pl.ds(rbase, _RT)])


def _p7_body(p_ref, g_ref, rt_ref, b_ref, o_ref):
    s = p_ref[0] + p_ref[1]
    r = 1.0 / jnp.maximum(s, 1e-10)
    rex = jnp.dot(r, rt_ref[...], preferred_element_type=f32)
    o_ref[...] = (g_ref[0] + g_ref[1]) * rex + b_ref[...]


def _p7(p, agg, rtmat, bias2):
    return pl.pallas_call(
        _p7_body,
        grid=(N // _BN,),
        in_specs=[
            pl.BlockSpec((NC, _BN, 16), lambda i: (0, i, 0)),
            pl.BlockSpec((NC, _BN, HF), lambda i: (0, i, 0)),
            pl.BlockSpec((16, HF), lambda i: (0, 0)),
            pl.BlockSpec((1, HF), lambda i: (0, 0)),
        ],
        out_specs=pl.BlockSpec((_BN, HF), lambda i: (i, 0)),
        out_shape=jax.ShapeDtypeStruct((N, HF), f32),
    )(p, agg, rtmat, bias2)


def kernel(x, edge_index, edge_attr, W, a_src, a_dst, We, be, bias):
    src = edge_index[0].astype(i32)
    dst = edge_index[1].astype(i32)
    wt = W.T
    wet = We.T
    ar = jnp.arange(HF)
    hid = ar // OUT_F
    aexp = jnp.zeros((HF, 16), f32).at[ar, hid].set(a_src.reshape(-1))
    adst = jnp.zeros((HF, 16), f32).at[ar, hid].set(a_dst.reshape(-1))
    smat = (hid[:, None] == jnp.arange(16)[None, :]).astype(f32)
    rtmat = smat.T
    be2 = be.reshape(1, HF)
    bias2 = bias.reshape(1, HF)
    z16 = jnp.zeros((N, 16), f32)
    z128 = jnp.zeros((N, HF), f32)

    h, asrc16 = _p1(x, wt, aexp)
    hd, ase = _p2(h, asrc16, src, dst)
    expsc = _p3(edge_attr, hd, ase, wet, be2, adst, smat)
    p, agg = _p6m(h, expsc, src, dst, z16, z128)
    return _p7(p, agg, rtmat, bias2)


# speedup vs baseline: 1.0120x; 1.0120x over previous
import functools

import jax
import jax.numpy as jnp
from jax import lax
from jax.experimental import pallas as pl
from jax.experimental.pallas import tpu as pltpu
from jax.experimental.pallas import tpu_sc as plsc

N = 10000
E = 320000
EH = E // 2
HEADS = 8
OUT_F = 16
HF = HEADS * OUT_F

NC = 2
NS = 16
NW = NC * NS

f32 = jnp.float32
i32 = jnp.int32

_mesh = plsc.VectorSubcoreMesh(core_axis_name="c", subcore_axis_name="s")

_BN = 1000


def _p1_body(x_ref, wt_ref, a_ref, h_ref, as_ref):
    h = jnp.dot(x_ref[...], wt_ref[...], preferred_element_type=f32)
    h_ref[...] = h
    as_ref[...] = jnp.dot(h, a_ref[...], preferred_element_type=f32)


def _p1(x, wt, aexp):
    return pl.pallas_call(
        _p1_body,
        grid=(N // _BN,),
        in_specs=[
            pl.BlockSpec((_BN, HF), lambda i: (i, 0)),
            pl.BlockSpec((HF, HF), lambda i: (0, 0)),
            pl.BlockSpec((HF, 16), lambda i: (0, 0)),
        ],
        out_specs=[
            pl.BlockSpec((_BN, HF), lambda i: (i, 0)),
            pl.BlockSpec((_BN, 16), lambda i: (i, 0)),
        ],
        out_shape=[
            jax.ShapeDtypeStruct((N, HF), f32),
            jax.ShapeDtypeStruct((N, 16), f32),
        ],
    )(x, wt, aexp)


_CA = 256
_NCH_A = EH // _CA
_ITER_A = (_NCH_A + NW - 1) // NW


@functools.partial(
    pl.kernel,
    out_type=jax.ShapeDtypeStruct((EH, HF), f32),
    mesh=_mesh,
    scratch_types=[
        pltpu.VMEM((_CA,), i32),
        pltpu.VMEM((_CA, HF), f32),
        pltpu.SemaphoreType.DMA,
    ],
)
def _p2a(h_hbm, dst_hbm, hd_out, di_v, hd_v, sem1):
    wid = lax.axis_index("s") * NC + lax.axis_index("c")

    def chunk(ci, carry):
        ck = wid + NW * ci

        @pl.when(ck < _NCH_A)
        def _():
            base = ck * _CA
            pltpu.sync_copy(dst_hbm.at[pl.ds(base, _CA)], di_v)
            for g in range(_CA // 128):
                sl = pl.ds(g * 128, 128)
                pltpu.async_copy(h_hbm.at[di_v.at[sl]], hd_v.at[sl], sem1)
            for g in range(_CA // 128):
                sl = pl.ds(g * 128, 128)
                pltpu.make_async_copy(h_hbm.at[di_v.at[sl]], hd_v.at[sl], sem1).wait()
            pltpu.sync_copy(hd_v, hd_out.at[pl.ds(base, _CA)])

        return carry

    lax.fori_loop(0, _ITER_A, chunk, None)


_CB = 512
_NCH_B = E // _CB
_ITER_B = (_NCH_B + NW - 1) // NW


@functools.partial(
    pl.kernel,
    out_type=jax.ShapeDtypeStruct((E, 16), f32),
    mesh=_mesh,
    compiler_params=pltpu.CompilerParams(use_tc_tiling_on_sc=False, needs_layout_passes=False),
    scratch_types=[
        pltpu.VMEM((_CB,), i32),
        pltpu.VMEM((_CB, 16), f32),
        pltpu.SemaphoreType.DMA,
    ],
)
def _p2b(as_hbm, src_hbm, ase_out, si_v, ase_v, sem2):
    wid = lax.axis_index("s") * NC + lax.axis_index("c")

    def chunk(ci, carry):
        ck = wid + NW * ci

        @pl.when(ck < _NCH_B)
        def _():
            base = ck * _CB
            pltpu.sync_copy(src_hbm.at[pl.ds(base, _CB)], si_v)
            for g in range(_CB // 128):
                sl = pl.ds(g * 128, 128)
                pltpu.async_copy(as_hbm.at[si_v.at[sl]], ase_v.at[sl], sem2)
            for g in range(_CB // 128):
                sl = pl.ds(g * 128, 128)
                pltpu.make_async_copy(as_hbm.at[si_v.at[sl]], ase_v.at[sl], sem2).wait()
            pltpu.sync_copy(ase_v, ase_out.at[pl.ds(base, _CB)])

        return carry

    lax.fori_loop(0, _ITER_B, chunk, None)


_BE = 3200
_BP = _BE // 8


def _p3_body(ea_ref, ase_ref, h0, h1, h2, h3, h4, h5, h6, h7,
             wet_ref, be_ref, adst_ref, s_ref, out_ref):
    ea = ea_ref[...]
    ase = ase_ref[...]
    hrefs = (h0, h1, h2, h3, h4, h5, h6, h7)
    outs = []
    for m in range(8):
        sl = slice(16 * m, 16 * (m + 1))
        hd_m = hrefs[m][0]
        enc = jnp.maximum(
            jnp.dot(ea[:, sl], wet_ref[...], preferred_element_type=f32)
            + be_ref[...], 0.0)
        aenc = jnp.dot(enc * hd_m, s_ref[...], preferred_element_type=f32)
        ad = jnp.dot(hd_m, adst_ref[...], preferred_element_type=f32)
        alpha = ase[:, sl] + aenc + ad
        alpha = jnp.where(alpha > 0, alpha, 0.2 * alpha)
        outs.append(jnp.exp(alpha))
    out_ref[...] = jnp.concatenate(outs, axis=1)


def _p3(ea2, ase2, hd3, wet, be2, adst, smat):
    ep = ea2.shape[0]
    hd_specs = [
        pl.BlockSpec((1, _BP, HF), (lambda i, m=m: (m, i, 0))) for m in range(8)
    ]
    return pl.pallas_call(
        _p3_body,
        grid=(ep // _BP,),
        in_specs=[
            pl.BlockSpec((_BP, HF), lambda i: (i, 0)),
            pl.BlockSpec((_BP, HF), lambda i: (i, 0)),
            *hd_specs,
            pl.BlockSpec((16, HF), lambda i: (0, 0)),
            pl.BlockSpec((1, HF), lambda i: (0, 0)),
            pl.BlockSpec((HF, 16), lambda i: (0, 0)),
            pl.BlockSpec((HF, 16), lambda i: (0, 0)),
        ],
        out_specs=pl.BlockSpec((_BP, HF), lambda i: (i, 0)),
        out_shape=jax.ShapeDtypeStruct((ep, HF), f32),
    )(ea2, ase2, *[hd3] * 8, wet, be2, adst, smat)


_C6 = 64
_K6 = 4
_KI = 8
_ESC = EH // NC
_NCH_6 = _ESC // _C6
_CMAX = -(-_NCH_6 // NS)
_NOCT = (_CMAX + _KI - 1) // _KI
_RT = N // NS
_NG = _C6 // 16


@functools.partial(
    pl.kernel,
    out_type=(
        jax.ShapeDtypeStruct((NC, N, 16), f32),
        jax.ShapeDtypeStruct((NC, N, HF), f32),
    ),
    mesh=_mesh,
    compiler_params=pltpu.CompilerParams(use_tc_tiling_on_sc=False, needs_layout_passes=False),
    scratch_types=[
        pltpu.VMEM((_KI, _C6), i32),
        pltpu.VMEM((_KI, _C6), i32),
        pltpu.VMEM((_K6, _C6, HF), f32),
        pltpu.VMEM((_K6, _C6, 16), f32),
        pltpu.VMEM_SHARED((N, 16), f32),
        pltpu.VMEM_SHARED((N, HF), f32),
        pltpu.SemaphoreType.DMA,
        pltpu.SemaphoreType.DMA,
        pltpu.SemaphoreType.DMA,
        pltpu.SemaphoreType.DMA,
        pltpu.SemaphoreType.DMA,
    ],
)
def _p6m(h_hbm, exp_hbm, src_hbm, dst_hbm, z16_hbm, z128_hbm,
         p_out, agg_out,
         si_v, di_v, hs_v, e_v,
         acc16, acc128, sem_i, sem_h, sem_e, sem_s16, sem_s128):
    cid = lax.axis_index("c")
    sid = lax.axis_index("s")
    rbase = sid * _RT
    pltpu.sync_copy(z16_hbm.at[pl.ds(rbase, _RT)], acc16.at[pl.ds(rbase, _RT)])
    pltpu.sync_copy(z128_hbm.at[pl.ds(rbase, _RT)], acc128.at[pl.ds(rbase, _RT)])
    plsc.subcore_barrier()

    def ck_of(c):
        return sid + NS * c

    def valid(c):
        return ck_of(c) < _NCH_6

    def fire_idx(c, s8):
        base = cid * _ESC + ck_of(c) * _C6
        pltpu.async_copy(src_hbm.at[pl.ds(base, _C6)], si_v.at[s8], sem_i)
        pltpu.async_copy(dst_hbm.at[pl.ds(base, _C6)], di_v.at[s8], sem_i)

    def wait_idx(s8):
        pltpu.make_async_copy(src_hbm.at[pl.ds(0, _C6)], si_v.at[s8], sem_i).wait()
        pltpu.make_async_copy(dst_hbm.at[pl.ds(0, _C6)], di_v.at[s8], sem_i).wait()

    def fire_main(c, s8, s4):
        base = cid * _ESC + ck_of(c) * _C6
        pltpu.async_copy(h_hbm.at[si_v.at[s8]], hs_v.at[s4], sem_h)
        pltpu.async_copy(exp_hbm.at[pl.ds(base, _C6)], e_v.at[s4], sem_e)

    def wait_main(s8, s4):
        pltpu.make_async_copy(h_hbm.at[si_v.at[s8]], hs_v.at[s4], sem_h).wait()
        pltpu.make_async_copy(exp_hbm.at[pl.ds(0, _C6)], e_v.at[s4], sem_e).wait()

    def wait_scatters(s4):
        pltpu.make_async_copy(e_v.at[s4], acc16.at[di_v.at[0]], sem_s16).wait()
        pltpu.make_async_copy(hs_v.at[s4], acc128.at[di_v.at[0]], sem_s128).wait()

    def process(s8, s4):
        pltpu.async_copy(e_v.at[s4], acc16.at[di_v.at[s8]], sem_s16, add=True)

        def edge(ei, carry2):
            ev_row = e_v.at[s4][ei, :]
            for hh_ in range(HEADS):
                w = ev_row[jnp.full((16,), hh_, i32)]
                sl = pl.ds(hh_ * OUT_F, OUT_F)
                hs_v.at[s4][ei, sl] = hs_v.at[s4][ei, sl] * w
            return carry2

        lax.fori_loop(0, _C6, edge, None)
        pltpu.async_copy(hs_v.at[s4], acc128.at[di_v.at[s8]], sem_s128, add=True)

    fire_idx(0, 0)
    fire_idx(1, 1)
    wait_idx(0)
    fire_main(0, 0, 0)

    def octet(j, carry):
        for par in range(_KI):
            c = _KI * j + par
            s8 = par
            n8 = (par + 1) % _KI
            s4 = par % _K6
            n4 = (par + 1) % _K6

            @pl.when(valid(c + 1))
            def _():
                wait_idx(n8)

                @pl.when(c + 1 >= _K6)
                def _():
                    wait_scatters(n4)

                fire_main(c + 1, n8, n4)

            @pl.when(valid(c))
            def _():
                wait_main(s8, s4)
                process(s8, s4)

            @pl.when(valid(c + 2))
            def _():
                fire_idx(c + 2, (par + 2) % _KI)

        return carry

    lax.fori_loop(0, _NOCT, octet, None)
    for s4 in range(_K6):
        @pl.when(ck_of(s4) < _NCH_6)
        def _():
            wait_scatters(s4)

    plsc.subcore_barrier()
    pltpu.sync_copy(acc16.at[pl.ds(rbase, _RT)],
                    p_out.at[cid].at[pl.ds(rbase, _RT)])
    pltpu.sync_copy(acc128.at[pl.ds(rbase, _RT)],
                    agg_out.at[cid].at[pl.ds(rbase, _RT)])


def _p7_body(p_ref, g_ref, rt_ref, b_ref, o_ref):
    s = p_ref[0] + p_ref[1] + p_ref[2] + p_ref[3]
    r = 1.0 / jnp.maximum(s, 1e-10)
    rex = jnp.dot(r, rt_ref[...], preferred_element_type=f32)
    g = g_ref[0] + g_ref[1] + g_ref[2] + g_ref[3]
    o_ref[...] = g * rex + b_ref[...]


def _p7(p, agg, rtmat, bias2):
    return pl.pallas_call(
        _p7_body,
        grid=(N // _BN,),
        in_specs=[
            pl.BlockSpec((4, _BN, 16), lambda i: (0, i, 0)),
            pl.BlockSpec((4, _BN, HF), lambda i: (0, i, 0)),
            pl.BlockSpec((16, HF), lambda i: (0, 0)),
            pl.BlockSpec((1, HF), lambda i: (0, 0)),
        ],
        out_specs=pl.BlockSpec((_BN, HF), lambda i: (i, 0)),
        out_shape=jax.ShapeDtypeStruct((N, HF), f32),
    )(p, agg, rtmat, bias2)


def kernel(x, edge_index, edge_attr, W, a_src, a_dst, We, be, bias):
    src = edge_index[0].astype(i32)
    dst = edge_index[1].astype(i32)
    wt = W.T
    wet = We.T
    ar = jnp.arange(HF)
    hid = ar // OUT_F
    sel = (hid[:, None] == jnp.arange(16)[None, :]).astype(f32)
    aexp = sel * a_src.reshape(-1)[:, None]
    adst = sel * a_dst.reshape(-1)[:, None]
    smat = sel
    rtmat = sel.T
    be2 = be.reshape(1, HF)
    bias2 = bias.reshape(1, HF)
    z16 = jnp.zeros((N, 16), f32)
    z128 = jnp.zeros((N, HF), f32)

    h, asrc16 = _p1(x, wt, aexp)
    ase2 = _p2b(asrc16, src).reshape(E // 8, HF)
    ea2 = edge_attr.reshape(E // 8, HF)

    ps, aggs = [], []
    for half in range(2):
        e0 = half * EH
        dst_h = dst[e0:e0 + EH]
        dst_p = dst_h.reshape(EH // 8, 8).T.reshape(-1)
        hd = _p2a(h, dst_p)
        hd3 = hd.reshape(8, EH // 8, HF)
        exp2 = _p3(ea2[e0 // 8:(e0 + EH) // 8], ase2[e0 // 8:(e0 + EH) // 8],
                   hd3, wet, be2, adst, smat)
        expsc = exp2.reshape(EH, 16)
        p, agg = _p6m(h, expsc, src[e0:e0 + EH], dst_h, z16, z128)
        ps.append(p)
        aggs.append(agg)

    p4 = jnp.concatenate(ps, axis=0)
    agg4 = jnp.concatenate(aggs, axis=0)
    return _p7(p4, agg4, rtmat, bias2)


# speedup vs baseline: 1.4991x; 1.4813x over previous
import functools

import jax
import jax.numpy as jnp
from jax import lax
from jax.experimental import pallas as pl
from jax.experimental.pallas import tpu as pltpu
from jax.experimental.pallas import tpu_sc as plsc

N = 10000
E = 320000
EH = E // 2
HEADS = 8
OUT_F = 16
HF = HEADS * OUT_F

NC = 2
NS = 16
NW = NC * NS

f32 = jnp.float32
i32 = jnp.int32

_mesh = plsc.VectorSubcoreMesh(core_axis_name="c", subcore_axis_name="s")

_BN = 1000


def _p1_body(x_ref, wt_ref, a_ref, h_ref, as_ref):
    h = jnp.dot(x_ref[...], wt_ref[...], preferred_element_type=f32)
    h_ref[...] = h
    as_ref[...] = jnp.dot(h, a_ref[...], preferred_element_type=f32)


def _p1(x, wt, aexp):
    return pl.pallas_call(
        _p1_body,
        grid=(N // _BN,),
        in_specs=[
            pl.BlockSpec((_BN, HF), lambda i: (i, 0)),
            pl.BlockSpec((HF, HF), lambda i: (0, 0)),
            pl.BlockSpec((HF, 16), lambda i: (0, 0)),
        ],
        out_specs=[
            pl.BlockSpec((_BN, HF), lambda i: (i, 0)),
            pl.BlockSpec((_BN, 16), lambda i: (i, 0)),
        ],
        out_shape=[
            jax.ShapeDtypeStruct((N, HF), f32),
            jax.ShapeDtypeStruct((N, 16), f32),
        ],
    )(x, wt, aexp)


_CA = 256
_NCH_A = EH // _CA
_ITER_A = (_NCH_A + NW - 1) // NW


@functools.partial(
    pl.kernel,
    out_type=jax.ShapeDtypeStruct((EH, HF), f32),
    mesh=_mesh,
    scratch_types=[
        pltpu.VMEM((_CA,), i32),
        pltpu.VMEM((_CA, HF), f32),
        pltpu.SemaphoreType.DMA,
    ],
)
def _p2a(h_hbm, dst_hbm, hd_out, di_v, hd_v, sem1):
    wid = lax.axis_index("s") * NC + lax.axis_index("c")

    def chunk(ci, carry):
        ck = wid + NW * ci

        @pl.when(ck < _NCH_A)
        def _():
            base = ck * _CA
            pltpu.sync_copy(dst_hbm.at[pl.ds(base, _CA)], di_v)
            for g in range(_CA // 128):
                sl = pl.ds(g * 128, 128)
                pltpu.async_copy(h_hbm.at[di_v.at[sl]], hd_v.at[sl], sem1)
            for g in range(_CA // 128):
                sl = pl.ds(g * 128, 128)
                pltpu.make_async_copy(h_hbm.at[di_v.at[sl]], hd_v.at[sl], sem1).wait()
            pltpu.sync_copy(hd_v, hd_out.at[pl.ds(base, _CA)])

        return carry

    lax.fori_loop(0, _ITER_A, chunk, None)


_CB = 512
_NCH_B = E // _CB
_ITER_B = (_NCH_B + NW - 1) // NW


@functools.partial(
    pl.kernel,
    out_type=jax.ShapeDtypeStruct((E, 16), f32),
    mesh=_mesh,
    compiler_params=pltpu.CompilerParams(use_tc_tiling_on_sc=False, needs_layout_passes=False),
    scratch_types=[
        pltpu.VMEM((_CB,), i32),
        pltpu.VMEM((_CB, 16), f32),
        pltpu.SemaphoreType.DMA,
    ],
)
def _p2b(as_hbm, src_hbm, ase_out, si_v, ase_v, sem2):
    wid = lax.axis_index("s") * NC + lax.axis_index("c")

    def chunk(ci, carry):
        ck = wid + NW * ci

        @pl.when(ck < _NCH_B)
        def _():
            base = ck * _CB
            pltpu.sync_copy(src_hbm.at[pl.ds(base, _CB)], si_v)
            for g in range(_CB // 128):
                sl = pl.ds(g * 128, 128)
                pltpu.async_copy(as_hbm.at[si_v.at[sl]], ase_v.at[sl], sem2)
            for g in range(_CB // 128):
                sl = pl.ds(g * 128, 128)
                pltpu.make_async_copy(as_hbm.at[si_v.at[sl]], ase_v.at[sl], sem2).wait()
            pltpu.sync_copy(ase_v, ase_out.at[pl.ds(base, _CB)])

        return carry

    lax.fori_loop(0, _ITER_B, chunk, None)


_BE = 3200
_BP = _BE // 8


def _p3_body(ea_ref, ase_ref, h0, h1, h2, h3, h4, h5, h6, h7,
             wet_ref, be_ref, adst_ref, s_ref, out_ref):
    ea = ea_ref[...]
    ase = ase_ref[...]
    hrefs = (h0, h1, h2, h3, h4, h5, h6, h7)
    outs = []
    for m in range(8):
        sl = slice(16 * m, 16 * (m + 1))
        hd_m = hrefs[m][0]
        enc = jnp.maximum(
            jnp.dot(ea[:, sl], wet_ref[...], preferred_element_type=f32)
            + be_ref[...], 0.0)
        aenc = jnp.dot(enc * hd_m, s_ref[...], preferred_element_type=f32)
        ad = jnp.dot(hd_m, adst_ref[...], preferred_element_type=f32)
        alpha = ase[:, sl] + aenc + ad
        alpha = jnp.where(alpha > 0, alpha, 0.2 * alpha)
        outs.append(jnp.exp(alpha))
    out_ref[...] = jnp.concatenate(outs, axis=1)


def _p3(ea2, ase2, hd3, wet, be2, adst, smat, off_blocks):
    hd_specs = [
        pl.BlockSpec((1, _BP, HF), (lambda i, m=m: (m, i, 0))) for m in range(8)
    ]
    return pl.pallas_call(
        _p3_body,
        grid=(EH // _BE,),
        in_specs=[
            pl.BlockSpec((_BP, HF), lambda i: (i + off_blocks, 0)),
            pl.BlockSpec((_BP, HF), lambda i: (i + off_blocks, 0)),
            *hd_specs,
            pl.BlockSpec((16, HF), lambda i: (0, 0)),
            pl.BlockSpec((1, HF), lambda i: (0, 0)),
            pl.BlockSpec((HF, 16), lambda i: (0, 0)),
            pl.BlockSpec((HF, 16), lambda i: (0, 0)),
        ],
        out_specs=pl.BlockSpec((_BP, HF), lambda i: (i, 0)),
        out_shape=jax.ShapeDtypeStruct((EH // 8, HF), f32),
    )(ea2, ase2, *[hd3] * 8, wet, be2, adst, smat)


_C6 = 64
_K6 = 4
_KI = 8
_ESC = EH // NC
_NCH_6 = _ESC // _C6
_CMAX = -(-_NCH_6 // NS)
_NOCT = (_CMAX + _KI - 1) // _KI
_RT = N // NS
_NG = _C6 // 16


def _make_p6m(eoff):
  @functools.partial(
    pl.kernel,
    out_type=(
        jax.ShapeDtypeStruct((NC, N, 16), f32),
        jax.ShapeDtypeStruct((NC, N, HF), f32),
    ),
    mesh=_mesh,
    compiler_params=pltpu.CompilerParams(use_tc_tiling_on_sc=False, needs_layout_passes=False),
    scratch_types=[
        pltpu.VMEM((_KI, _C6), i32),
        pltpu.VMEM((_KI, _C6), i32),
        pltpu.VMEM((_K6, _C6, HF), f32),
        pltpu.VMEM((_K6, _C6, 16), f32),
        pltpu.VMEM_SHARED((N, 16), f32),
        pltpu.VMEM_SHARED((N, HF), f32),
        pltpu.SemaphoreType.DMA,
        pltpu.SemaphoreType.DMA,
        pltpu.SemaphoreType.DMA,
        pltpu.SemaphoreType.DMA,
        pltpu.SemaphoreType.DMA,
    ],
  )
  def _p6m(h_hbm, exp_hbm, src_hbm, dst_hbm, z16_hbm, z128_hbm,
           p_out, agg_out,
           si_v, di_v, hs_v, e_v,
           acc16, acc128, sem_i, sem_h, sem_e, sem_s16, sem_s128):
      cid = lax.axis_index("c")
      sid = lax.axis_index("s")
      rbase = sid * _RT
      pltpu.sync_copy(z16_hbm.at[pl.ds(rbase, _RT)], acc16.at[pl.ds(rbase, _RT)])
      pltpu.sync_copy(z128_hbm.at[pl.ds(rbase, _RT)], acc128.at[pl.ds(rbase, _RT)])
      plsc.subcore_barrier()

      def ck_of(c):
          return sid + NS * c

      def valid(c):
          return ck_of(c) < _NCH_6

      def fire_idx(c, s8):
          base = cid * _ESC + ck_of(c) * _C6
          pltpu.async_copy(src_hbm.at[pl.ds(eoff + base, _C6)], si_v.at[s8], sem_i)
          pltpu.async_copy(dst_hbm.at[pl.ds(eoff + base, _C6)], di_v.at[s8], sem_i)

      def wait_idx(s8):
          pltpu.make_async_copy(src_hbm.at[pl.ds(0, _C6)], si_v.at[s8], sem_i).wait()
          pltpu.make_async_copy(dst_hbm.at[pl.ds(0, _C6)], di_v.at[s8], sem_i).wait()

      def fire_main(c, s8, s4):
          base = cid * _ESC + ck_of(c) * _C6
          pltpu.async_copy(h_hbm.at[si_v.at[s8]], hs_v.at[s4], sem_h)
          pltpu.async_copy(exp_hbm.at[pl.ds(base, _C6)], e_v.at[s4], sem_e)

      def wait_main(s8, s4):
          pltpu.make_async_copy(h_hbm.at[si_v.at[s8]], hs_v.at[s4], sem_h).wait()
          pltpu.make_async_copy(exp_hbm.at[pl.ds(0, _C6)], e_v.at[s4], sem_e).wait()

      def wait_scatters(s4):
          pltpu.make_async_copy(e_v.at[s4], acc16.at[di_v.at[0]], sem_s16).wait()
          pltpu.make_async_copy(hs_v.at[s4], acc128.at[di_v.at[0]], sem_s128).wait()

      def process(s8, s4):
          pltpu.async_copy(e_v.at[s4], acc16.at[di_v.at[s8]], sem_s16, add=True)

          def edge(ei, carry2):
              ev_row = e_v.at[s4][ei, :]
              for hh_ in range(HEADS):
                  w = ev_row[jnp.full((16,), hh_, i32)]
                  sl = pl.ds(hh_ * OUT_F, OUT_F)
                  hs_v.at[s4][ei, sl] = hs_v.at[s4][ei, sl] * w
              return carry2

          lax.fori_loop(0, _C6, edge, None)
          pltpu.async_copy(hs_v.at[s4], acc128.at[di_v.at[s8]], sem_s128, add=True)

      fire_idx(0, 0)
      fire_idx(1, 1)
      wait_idx(0)
      fire_main(0, 0, 0)

      def octet(j, carry):
          for par in range(_KI):
              c = _KI * j + par
              s8 = par
              n8 = (par + 1) % _KI
              s4 = par % _K6
              n4 = (par + 1) % _K6

              @pl.when(valid(c + 1))
              def _():
                  wait_idx(n8)

                  @pl.when(c + 1 >= _K6)
                  def _():
                      wait_scatters(n4)

                  fire_main(c + 1, n8, n4)

              @pl.when(valid(c))
              def _():
                  wait_main(s8, s4)
                  process(s8, s4)

              @pl.when(valid(c + 2))
              def _():
                  fire_idx(c + 2, (par + 2) % _KI)

          return carry

      lax.fori_loop(0, _NOCT, octet, None)
      for s4 in range(_K6):
          @pl.when(ck_of(s4) < _NCH_6)
          def _():
              wait_scatters(s4)

      plsc.subcore_barrier()
      pltpu.sync_copy(acc16.at[pl.ds(rbase, _RT)],
                      p_out.at[cid].at[pl.ds(rbase, _RT)])
      pltpu.sync_copy(acc128.at[pl.ds(rbase, _RT)],
                      agg_out.at[cid].at[pl.ds(rbase, _RT)])

  return _p6m


_P6M = (_make_p6m(0), _make_p6m(EH))


def _p7_body(pa_ref, pb_ref, ga_ref, gb_ref, rt_ref, b_ref, o_ref):
    s = pa_ref[0] + pa_ref[1] + pb_ref[0] + pb_ref[1]
    r = 1.0 / jnp.maximum(s, 1e-10)
    rex = jnp.dot(r, rt_ref[...], preferred_element_type=f32)
    g = ga_ref[0] + ga_ref[1] + gb_ref[0] + gb_ref[1]
    o_ref[...] = g * rex + b_ref[...]


def _p7(pa, pb, ga, gb, rtmat, bias2):
    return pl.pallas_call(
        _p7_body,
        grid=(N // _BN,),
        in_specs=[
            pl.BlockSpec((NC, _BN, 16), lambda i: (0, i, 0)),
            pl.BlockSpec((NC, _BN, 16), lambda i: (0, i, 0)),
            pl.BlockSpec((NC, _BN, HF), lambda i: (0, i, 0)),
            pl.BlockSpec((NC, _BN, HF), lambda i: (0, i, 0)),
            pl.BlockSpec((16, HF), lambda i: (0, 0)),
            pl.BlockSpec((1, HF), lambda i: (0, 0)),
        ],
        out_specs=pl.BlockSpec((_BN, HF), lambda i: (i, 0)),
        out_shape=jax.ShapeDtypeStruct((N, HF), f32),
    )(pa, pb, ga, gb, rtmat, bias2)


def kernel(x, edge_index, edge_attr, W, a_src, a_dst, We, be, bias):
    src = edge_index[0].astype(i32)
    dst = edge_index[1].astype(i32)
    wt = W.T
    wet = We.T
    ar = jnp.arange(HF)
    hid = ar // OUT_F
    sel = (hid[:, None] == jnp.arange(16)[None, :]).astype(f32)
    aexp = sel * a_src.reshape(-1)[:, None]
    adst = sel * a_dst.reshape(-1)[:, None]
    smat = sel
    rtmat = sel.T
    be2 = be.reshape(1, HF)
    bias2 = bias.reshape(1, HF)
    z16 = jnp.zeros((N, 16), f32)
    z128 = jnp.zeros((N, HF), f32)

    h, asrc16 = _p1(x, wt, aexp)
    ase2 = _p2b(asrc16, src).reshape(E // 8, HF)
    ea2 = edge_attr.reshape(E // 8, HF)
    dst_t = dst.reshape(2, EH // 8, 8)

    ps, aggs = [], []
    for half in range(2):
        dst_p = dst_t[half].T.reshape(-1)
        hd = _p2a(h, dst_p)
        hd3 = hd.reshape(8, EH // 8, HF)
        exp2 = _p3(ea2, ase2, hd3, wet, be2, adst, smat,
                   half * (EH // 8 // _BP))
        expsc = exp2.reshape(EH, 16)
        p, agg = _P6M[half](h, expsc, src, dst, z16, z128)
        ps.append(p)
        aggs.append(agg)

    return _p7(ps[0], ps[1], aggs[0], aggs[1], rtmat, bias2)


# speedup vs baseline: 1.5696x; 1.0470x over previous
import functools

import jax
import jax.numpy as jnp
from jax import lax
from jax.experimental import pallas as pl
from jax.experimental.pallas import tpu as pltpu
from jax.experimental.pallas import tpu_sc as plsc

N = 10000
E = 320000
EH = E // 2
HEADS = 8
OUT_F = 16
HF = HEADS * OUT_F

NC = 2
NS = 16
NW = NC * NS

f32 = jnp.float32
i32 = jnp.int32

_mesh = plsc.VectorSubcoreMesh(core_axis_name="c", subcore_axis_name="s")

_BN = 1000


def _p1_body(x_ref, wt_ref, a_ref, h_ref, as_ref):
    h = jnp.dot(x_ref[...], wt_ref[...], preferred_element_type=f32)
    h_ref[...] = h
    as_ref[...] = jnp.dot(h, a_ref[...], preferred_element_type=f32)


def _p1(x, wt, aexp):
    return pl.pallas_call(
        _p1_body,
        grid=(N // _BN,),
        in_specs=[
            pl.BlockSpec((_BN, HF), lambda i: (i, 0)),
            pl.BlockSpec((HF, HF), lambda i: (0, 0)),
            pl.BlockSpec((HF, 16), lambda i: (0, 0)),
        ],
        out_specs=[
            pl.BlockSpec((_BN, HF), lambda i: (i, 0)),
            pl.BlockSpec((_BN, 16), lambda i: (i, 0)),
        ],
        out_shape=[
            jax.ShapeDtypeStruct((N, HF), f32),
            jax.ShapeDtypeStruct((N, 16), f32),
        ],
    )(x, wt, aexp)


_CA = 256
_NCH_A = EH // _CA
_ITER_A = (_NCH_A + NW - 1) // NW


def _make_p2a(eoff):
  @functools.partial(
    pl.kernel,
    out_type=jax.ShapeDtypeStruct((EH, HF), f32),
    mesh=_mesh,
    scratch_types=[
        pltpu.VMEM((_CA,), i32),
        pltpu.VMEM((_CA, HF), f32),
        pltpu.SemaphoreType.DMA,
    ],
  )
  def _p2a(h_hbm, dst_hbm, hd_out, di_v, hd_v, sem1):
    wid = lax.axis_index("s") * NC + lax.axis_index("c")

    def chunk(ci, carry):
        ck = wid + NW * ci

        @pl.when(ck < _NCH_A)
        def _():
            base = ck * _CA
            pltpu.sync_copy(dst_hbm.at[pl.ds(eoff + base, _CA)], di_v)
            for g in range(_CA // 128):
                sl = pl.ds(g * 128, 128)
                pltpu.async_copy(h_hbm.at[di_v.at[sl]], hd_v.at[sl], sem1)
            for g in range(_CA // 128):
                sl = pl.ds(g * 128, 128)
                pltpu.make_async_copy(h_hbm.at[di_v.at[sl]], hd_v.at[sl], sem1).wait()
            pltpu.sync_copy(hd_v, hd_out.at[pl.ds(base, _CA)])

        return carry

    lax.fori_loop(0, _ITER_A, chunk, None)

  return _p2a


_P2A = (_make_p2a(0), _make_p2a(EH))


_CB = 512
_NCH_B = E // _CB
_ITER_B = (_NCH_B + NW - 1) // NW


@functools.partial(
    pl.kernel,
    out_type=jax.ShapeDtypeStruct((E, 16), f32),
    mesh=_mesh,
    compiler_params=pltpu.CompilerParams(use_tc_tiling_on_sc=False, needs_layout_passes=False),
    scratch_types=[
        pltpu.VMEM((_CB,), i32),
        pltpu.VMEM((_CB, 16), f32),
        pltpu.SemaphoreType.DMA,
    ],
)
def _p2b(as_hbm, src_hbm, ase_out, si_v, ase_v, sem2):
    wid = lax.axis_index("s") * NC + lax.axis_index("c")

    def chunk(ci, carry):
        ck = wid + NW * ci

        @pl.when(ck < _NCH_B)
        def _():
            base = ck * _CB
            pltpu.sync_copy(src_hbm.at[pl.ds(base, _CB)], si_v)
            for g in range(_CB // 128):
                sl = pl.ds(g * 128, 128)
                pltpu.async_copy(as_hbm.at[si_v.at[sl]], ase_v.at[sl], sem2)
            for g in range(_CB // 128):
                sl = pl.ds(g * 128, 128)
                pltpu.make_async_copy(as_hbm.at[si_v.at[sl]], ase_v.at[sl], sem2).wait()
            pltpu.sync_copy(ase_v, ase_out.at[pl.ds(base, _CB)])

        return carry

    lax.fori_loop(0, _ITER_B, chunk, None)


_BE = 3200
_BP = _BE // 8


def _p3_body(ea_ref, ase_ref, h0, h1, h2, h3, h4, h5, h6, h7,
             wet_ref, be_ref, adst_ref, s_ref, out_ref):
    ea = ea_ref[...]
    ase = ase_ref[...]
    hrefs = (h0, h1, h2, h3, h4, h5, h6, h7)
    outs = []
    for m in range(8):
        sl = slice(16 * m, 16 * (m + 1))
        hd_m = hrefs[m][0]
        enc = jnp.maximum(
            jnp.dot(ea[:, sl], wet_ref[...], preferred_element_type=f32)
            + be_ref[...], 0.0)
        aenc = jnp.dot(enc * hd_m, s_ref[...], preferred_element_type=f32)
        ad = jnp.dot(hd_m, adst_ref[...], preferred_element_type=f32)
        alpha = ase[:, sl] + aenc + ad
        alpha = jnp.where(alpha > 0, alpha, 0.2 * alpha)
        outs.append(jnp.exp(alpha))
    out_ref[...] = jnp.concatenate(outs, axis=1)


def _p3(ea2, ase2, hd3, wet, be2, adst, smat, off_blocks):
    hd_specs = [
        pl.BlockSpec((1, _BP, HF), (lambda i, m=m: (m, i, 0))) for m in range(8)
    ]
    return pl.pallas_call(
        _p3_body,
        grid=(EH // _BE,),
        in_specs=[
            pl.BlockSpec((_BP, HF), lambda i: (i + off_blocks, 0)),
            pl.BlockSpec((_BP, HF), lambda i: (i + off_blocks, 0)),
            *hd_specs,
            pl.BlockSpec((16, HF), lambda i: (0, 0)),
            pl.BlockSpec((1, HF), lambda i: (0, 0)),
            pl.BlockSpec((HF, 16), lambda i: (0, 0)),
            pl.BlockSpec((HF, 16), lambda i: (0, 0)),
        ],
        out_specs=pl.BlockSpec((_BP, HF), lambda i: (i, 0)),
        out_shape=jax.ShapeDtypeStruct((EH // 8, HF), f32),
    )(ea2, ase2, *[hd3] * 8, wet, be2, adst, smat)


_C6 = 64
_K6 = 4
_KI = 8
_ESC = EH // NC
_NCH_6 = _ESC // _C6
_CMAX = -(-_NCH_6 // NS)
_NOCT = (_CMAX + _KI - 1) // _KI
_RT = N // NS
_NG = _C6 // 16


def _make_p6m(eoff):
  @functools.partial(
    pl.kernel,
    out_type=(
        jax.ShapeDtypeStruct((NC, N, 16), f32),
        jax.ShapeDtypeStruct((NC, N, HF), f32),
    ),
    mesh=_mesh,
    compiler_params=pltpu.CompilerParams(use_tc_tiling_on_sc=False, needs_layout_passes=False),
    scratch_types=[
        pltpu.VMEM((_KI, _C6), i32),
        pltpu.VMEM((_KI, _C6), i32),
        pltpu.VMEM((_K6, _C6, HF), f32),
        pltpu.VMEM((_K6, _C6, 16), f32),
        pltpu.VMEM_SHARED((N, 16), f32),
        pltpu.VMEM_SHARED((N, HF), f32),
        pltpu.SemaphoreType.DMA,
        pltpu.SemaphoreType.DMA,
        pltpu.SemaphoreType.DMA,
        pltpu.SemaphoreType.DMA,
        pltpu.SemaphoreType.DMA,
    ],
  )
  def _p6m(h_hbm, exp_hbm, src_hbm, dst_hbm, z16_hbm, z128_hbm,
           p_out, agg_out,
           si_v, di_v, hs_v, e_v,
           acc16, acc128, sem_i, sem_h, sem_e, sem_s16, sem_s128):
      cid = lax.axis_index("c")
      sid = lax.axis_index("s")
      rbase = sid * _RT
      pltpu.sync_copy(z16_hbm.at[pl.ds(rbase, _RT)], acc16.at[pl.ds(rbase, _RT)])
      pltpu.sync_copy(z128_hbm.at[pl.ds(rbase, _RT)], acc128.at[pl.ds(rbase, _RT)])
      plsc.subcore_barrier()

      def ck_of(c):
          return sid + NS * c

      def valid(c):
          return ck_of(c) < _NCH_6

      def fire_idx(c, s8):
          base = cid * _ESC + ck_of(c) * _C6
          pltpu.async_copy(src_hbm.at[pl.ds(eoff + base, _C6)], si_v.at[s8], sem_i)
          pltpu.async_copy(dst_hbm.at[pl.ds(eoff + base, _C6)], di_v.at[s8], sem_i)

      def wait_idx(s8):
          pltpu.make_async_copy(src_hbm.at[pl.ds(0, _C6)], si_v.at[s8], sem_i).wait()
          pltpu.make_async_copy(dst_hbm.at[pl.ds(0, _C6)], di_v.at[s8], sem_i).wait()

      def fire_main(c, s8, s4):
          base = cid * _ESC + ck_of(c) * _C6
          pltpu.async_copy(h_hbm.at[si_v.at[s8]], hs_v.at[s4], sem_h)
          pltpu.async_copy(exp_hbm.at[pl.ds(base, _C6)], e_v.at[s4], sem_e)

      def wait_main(s8, s4):
          pltpu.make_async_copy(h_hbm.at[si_v.at[s8]], hs_v.at[s4], sem_h).wait()
          pltpu.make_async_copy(exp_hbm.at[pl.ds(0, _C6)], e_v.at[s4], sem_e).wait()

      def wait_scatters(s4):
          pltpu.make_async_copy(e_v.at[s4], acc16.at[di_v.at[0]], sem_s16).wait()
          pltpu.make_async_copy(hs_v.at[s4], acc128.at[di_v.at[0]], sem_s128).wait()

      def process(s8, s4):
          pltpu.async_copy(e_v.at[s4], acc16.at[di_v.at[s8]], sem_s16, add=True)

          def edge(ei, carry2):
              ev_row = e_v.at[s4][ei, :]
              for hh_ in range(HEADS):
                  w = ev_row[jnp.full((16,), hh_, i32)]
                  sl = pl.ds(hh_ * OUT_F, OUT_F)
                  hs_v.at[s4][ei, sl] = hs_v.at[s4][ei, sl] * w
              return carry2

          lax.fori_loop(0, _C6, edge, None)
          pltpu.async_copy(hs_v.at[s4], acc128.at[di_v.at[s8]], sem_s128, add=True)

      fire_idx(0, 0)
      fire_idx(1, 1)
      wait_idx(0)
      fire_main(0, 0, 0)

      def octet(j, carry):
          for par in range(_KI):
              c = _KI * j + par
              s8 = par
              n8 = (par + 1) % _KI
              s4 = par % _K6
              n4 = (par + 1) % _K6

              @pl.when(valid(c + 1))
              def _():
                  wait_idx(n8)

                  @pl.when(c + 1 >= _K6)
                  def _():
                      wait_scatters(n4)

                  fire_main(c + 1, n8, n4)

              @pl.when(valid(c))
              def _():
                  wait_main(s8, s4)
                  process(s8, s4)

              @pl.when(valid(c + 2))
              def _():
                  fire_idx(c + 2, (par + 2) % _KI)

          return carry

      lax.fori_loop(0, _NOCT, octet, None)
      for s4 in range(_K6):
          @pl.when(ck_of(s4) < _NCH_6)
          def _():
              wait_scatters(s4)

      plsc.subcore_barrier()
      pltpu.sync_copy(acc16.at[pl.ds(rbase, _RT)],
                      p_out.at[cid].at[pl.ds(rbase, _RT)])
      pltpu.sync_copy(acc128.at[pl.ds(rbase, _RT)],
                      agg_out.at[cid].at[pl.ds(rbase, _RT)])

  return _p6m


_P6M = (_make_p6m(0), _make_p6m(EH))


def _p7_body(pa_ref, pb_ref, ga_ref, gb_ref, rt_ref, b_ref, o_ref):
    s = pa_ref[0] + pa_ref[1] + pb_ref[0] + pb_ref[1]
    r = 1.0 / jnp.maximum(s, 1e-10)
    rex = jnp.dot(r, rt_ref[...], preferred_element_type=f32)
    g = ga_ref[0] + ga_ref[1] + gb_ref[0] + gb_ref[1]
    o_ref[...] = g * rex + b_ref[...]


def _p7(pa, pb, ga, gb, rtmat, bias2):
    return pl.pallas_call(
        _p7_body,
        grid=(N // _BN,),
        in_specs=[
            pl.BlockSpec((NC, _BN, 16), lambda i: (0, i, 0)),
            pl.BlockSpec((NC, _BN, 16), lambda i: (0, i, 0)),
            pl.BlockSpec((NC, _BN, HF), lambda i: (0, i, 0)),
            pl.BlockSpec((NC, _BN, HF), lambda i: (0, i, 0)),
            pl.BlockSpec((16, HF), lambda i: (0, 0)),
            pl.BlockSpec((1, HF), lambda i: (0, 0)),
        ],
        out_specs=pl.BlockSpec((_BN, HF), lambda i: (i, 0)),
        out_shape=jax.ShapeDtypeStruct((N, HF), f32),
    )(pa, pb, ga, gb, rtmat, bias2)


def kernel(x, edge_index, edge_attr, W, a_src, a_dst, We, be, bias):
    src = edge_index[0].astype(i32)
    dst = edge_index[1].astype(i32)
    wt = W.T
    wet = We.T
    ar = jnp.arange(HF)
    hid = ar // OUT_F
    sel = (hid[:, None] == jnp.arange(16)[None, :]).astype(f32)
    aexp = sel * a_src.reshape(-1)[:, None]
    adst = sel * a_dst.reshape(-1)[:, None]
    smat = sel
    rtmat = sel.T
    be2 = be.reshape(1, HF)
    bias2 = bias.reshape(1, HF)
    z16 = jnp.zeros((N, 16), f32)
    z128 = jnp.zeros((N, HF), f32)

    ea2 = edge_attr.reshape(E // 8, HF)
    dst_p = dst.reshape(2, EH // 8, 8).transpose(0, 2, 1).reshape(-1)
    h, asrc16 = _p1(x, wt, aexp)
    ase2 = _p2b(asrc16, src).reshape(E // 8, HF)

    ps, aggs = [], []
    for half in range(2):
        hd = _P2A[half](h, dst_p)
        hd3 = hd.reshape(8, EH // 8, HF)
        exp2 = _p3(ea2, ase2, hd3, wet, be2, adst, smat,
                   half * (EH // 8 // _BP))
        expsc = exp2.reshape(EH, 16)
        p, agg = _P6M[half](h, expsc, src, dst, z16, z128)
        ps.append(p)
        aggs.append(agg)

    return _p7(ps[0], ps[1], aggs[0], aggs[1], rtmat, bias2)
